# baseline (speedup 1.0000x reference)
import functools

import jax
import jax.numpy as jnp
from jax import lax
from jax.experimental import pallas as pl
from jax.experimental.pallas import tpu as pltpu
from jax.experimental.pallas import tpu_sc as plsc

N = 10000
E = 320000
D = 128
OUT = 64

NC = 2
NS = 16
NW = NC * NS

CHUNK = 128
CPT = 80
HCPT = 40
E_PAD = NW * CPT * CHUNK
N_ACC = 10112
ZROWS = N_ACC // NS
HIST_W = 16

BR = 1000


def _mesh():
    return plsc.VectorSubcoreMesh(core_axis_name="c", subcore_axis_name="s")


@functools.lru_cache(maxsize=None)
def _hist_kernel():
    @functools.partial(
        pl.kernel,
        out_type=(
            jax.ShapeDtypeStruct((NC, N_ACC, HIST_W), jnp.float32),
            jax.ShapeDtypeStruct((NC, N_ACC, HIST_W), jnp.float32),
        ),
        mesh=_mesh(),
        scratch_types=[
            pltpu.VMEM((CPT, CHUNK), jnp.int32),
            pltpu.VMEM((CPT, CHUNK), jnp.int32),
            pltpu.VMEM((CHUNK, HIST_W), jnp.float32),
            pltpu.VMEM_SHARED((N_ACC, HIST_W), jnp.float32),
            pltpu.VMEM_SHARED((N_ACC, HIST_W), jnp.float32),
        ],
    )
    def hist(src_hbm, dst_hbm, ones_hbm, z_hbm, degp_hbm, cntp_hbm,
             src_v, dst_v, ones_v, accd, accc):
        c = lax.axis_index("c")
        s = lax.axis_index("s")
        w = c * NS + s
        pltpu.sync_copy(z_hbm.at[pl.ds(s * ZROWS, ZROWS)],
                        accd.at[pl.ds(s * ZROWS, ZROWS)])
        pltpu.sync_copy(z_hbm.at[pl.ds(s * ZROWS, ZROWS)],
                        accc.at[pl.ds(s * ZROWS, ZROWS)])
        pltpu.sync_copy(ones_hbm, ones_v)
        pltpu.sync_copy(src_hbm.at[pl.ds(w * CPT, CPT)], src_v)
        pltpu.sync_copy(dst_hbm.at[pl.ds(w * CPT, CPT)], dst_v)
        plsc.subcore_barrier()

        def body(j, carry):
            pltpu.sync_copy(ones_v, accd.at[src_v.at[j]], add=True)
            pltpu.sync_copy(ones_v, accc.at[dst_v.at[j]], add=True)
            return carry

        lax.fori_loop(0, CPT, body, 0)
        plsc.subcore_barrier()
        pltpu.sync_copy(accd.at[pl.ds(s * ZROWS, ZROWS)],
                        degp_hbm.at[c, pl.ds(s * ZROWS, ZROWS)])
        pltpu.sync_copy(accc.at[pl.ds(s * ZROWS, ZROWS)],
                        cntp_hbm.at[c, pl.ds(s * ZROWS, ZROWS)])

    return hist


@functools.lru_cache(maxsize=None)
def _scatter_kernel():
    @functools.partial(
        pl.kernel,
        out_type=jax.ShapeDtypeStruct((NC, N_ACC, D), jnp.float32),
        mesh=_mesh(),
        scratch_types=[
            pltpu.VMEM((HCPT, CHUNK), jnp.int32),
            pltpu.VMEM((HCPT, CHUNK), jnp.int32),
            pltpu.VMEM((CHUNK, D), jnp.float32),
            pltpu.VMEM((CHUNK, D), jnp.float32),
            pltpu.VMEM_SHARED((N_ACC, D), jnp.float32),
            pltpu.SemaphoreType.DMA,
            pltpu.SemaphoreType.DMA,
        ],
    )
    def scat(src_hbm, dst_hbm, ym_hbm, z_hbm, out_hbm,
             src_v, dst_v, rows_a, rows_b, acc, sem_a, sem_b):
        c = lax.axis_index("c")
        s = lax.axis_index("s")
        w = c * NS + s
        pltpu.sync_copy(z_hbm.at[pl.ds(s * ZROWS, ZROWS)],
                        acc.at[pl.ds(s * ZROWS, ZROWS)])
        plsc.subcore_barrier()

        def body(i, carry):
            j = 2 * i
            da = pltpu.async_copy(ym_hbm.at[c].at[src_v.at[j]], rows_a, sem_a)
            db = pltpu.async_copy(ym_hbm.at[c].at[src_v.at[j + 1]], rows_b,
                                  sem_b)
            da.wait()
            pltpu.sync_copy(rows_a, acc.at[dst_v.at[j]], add=True)
            db.wait()
            pltpu.sync_copy(rows_b, acc.at[dst_v.at[j + 1]], add=True)
            return carry

        for h in range(CPT // HCPT):
            base = w * CPT + h * HCPT
            pltpu.sync_copy(src_hbm.at[pl.ds(base, HCPT)], src_v)
            pltpu.sync_copy(dst_hbm.at[pl.ds(base, HCPT)], dst_v)
            lax.fori_loop(0, HCPT // 2, body, 0)
        plsc.subcore_barrier()
        pltpu.sync_copy(acc.at[pl.ds(s * ZROWS, ZROWS)],
                        out_hbm.at[c, pl.ds(s * ZROWS, ZROWS)])

    return scat


def _dis_icnt(degp, cntp):
    deg = (degp[0] + degp[1])[:, :1]
    cnt = (cntp[0] + cntp[1])[:, :1]
    return lax.rsqrt(deg), 1.0 / cnt


def _combine(p, ym, xa, dis, icnt):
    s = p[0] + p[1]
    aggr = dis * (s + ym) * icnt
    o = jnp.maximum(aggr + xa, 0.0)
    n2 = jnp.sum(o * o, axis=1, keepdims=True)
    nrm = jnp.maximum(jnp.sqrt(n2), 1e-12)
    return o / nrm


def _tc_a_body(x_ref, W_ref, b_ref, Wa_ref, degp_ref, cntp_ref, ym_ref, xa_ref):
    xb = x_ref[...]
    dis, _ = _dis_icnt(degp_ref[...], cntp_ref[...])
    xm = jnp.maximum(jnp.dot(xb, W_ref[...],
                             preferred_element_type=jnp.float32) + b_ref[...], 0.0)
    ym = xm * dis
    ym_ref[0] = ym
    ym_ref[1] = ym
    xa_ref[...] = jnp.dot(xb, Wa_ref[...], preferred_element_type=jnp.float32)


def _tc_b_body(p_ref, ym_ref, xa_ref, degp_ref, cntp_ref, W_ref, b_ref, Wa_ref,
               ym2_ref, xa2_ref):
    dis, icnt = _dis_icnt(degp_ref[...], cntp_ref[...])
    h = _combine(p_ref[...], ym_ref[0], xa_ref[...], dis, icnt)
    xm2 = jnp.maximum(jnp.dot(h, W_ref[...],
                              preferred_element_type=jnp.float32) + b_ref[...], 0.0)
    ym2 = xm2 * dis
    ym2_ref[0] = ym2
    ym2_ref[1] = ym2
    xa2_ref[...] = jnp.dot(h, Wa_ref[...], preferred_element_type=jnp.float32)


def _tc_c_body(p_ref, ym_ref, xa_ref, degp_ref, cntp_ref,
               W1_ref, b1_ref, W2_ref, b2_ref, out_ref):
    dis, icnt = _dis_icnt(degp_ref[...], cntp_ref[...])
    h = _combine(p_ref[...], ym_ref[0], xa_ref[...], dis, icnt)
    z = jnp.dot(h, W1_ref[...], preferred_element_type=jnp.float32) + b1_ref[...]
    o = jnp.dot(z, W2_ref[...], preferred_element_type=jnp.float32) + b2_ref[...]
    m = jnp.max(o, axis=1, keepdims=True)
    lse = jnp.log(jnp.sum(jnp.exp(o - m), axis=1, keepdims=True)) + m
    out_ref[...] = o - lse


def _row_spec(rows, cols):
    return pl.BlockSpec((rows, cols), lambda i: (i, 0))


def _full_spec(shape):
    ndim = len(shape)
    return pl.BlockSpec(shape, lambda i, _n=ndim: (0,) * _n)


def _part_spec(width):
    return pl.BlockSpec((NC, BR, width), lambda i: (0, i, 0))


def kernel(x, edge_index, batch, lin1_W, lin1_b, agg1_W, lin2_W, lin2_b,
           agg2_W, mp1_W, mp1_b, mp2_W, mp2_b):
    src = edge_index[0]
    dst = edge_index[1]
    pad = E_PAD - E
    scratch_idx = N + jnp.arange(pad, dtype=jnp.int32) % (N_ACC - N)
    dst_p = jnp.concatenate([dst, scratch_idx]).reshape(NW * CPT, CHUNK)
    src_h = jnp.concatenate([src, scratch_idx]).reshape(NW * CPT, CHUNK)
    src_s = jnp.concatenate([src, jnp.zeros((pad,), jnp.int32)]).reshape(
        NW * CPT, CHUNK)
    ones_h = jnp.ones((CHUNK, HIST_W), jnp.float32)
    z_h = jnp.zeros((N_ACC, HIST_W), jnp.float32)
    z_d = jnp.zeros((N_ACC, D), jnp.float32)

    degp, cntp = _hist_kernel()(src_h, dst_p, ones_h, z_h)

    ym_shape = jax.ShapeDtypeStruct((NC, N_ACC, D), jnp.float32)
    grid = (N // BR,)
    ym1, xa1 = pl.pallas_call(
        _tc_a_body,
        grid=grid,
        in_specs=[
            _row_spec(BR, D), _full_spec((D, D)), _full_spec((1, D)),
            _full_spec((D, D)), _part_spec(HIST_W), _part_spec(HIST_W),
        ],
        out_specs=[_part_spec(D), _row_spec(BR, D)],
        out_shape=[ym_shape, jax.ShapeDtypeStruct((N, D), jnp.float32)],
    )(x, lin1_W, lin1_b.reshape(1, D), agg1_W, degp, cntp)

    p1 = _scatter_kernel()(src_s, dst_p, ym1, z_d)

    ym2, xa2 = pl.pallas_call(
        _tc_b_body,
        grid=grid,
        in_specs=[
            _part_spec(D), _part_spec(D), _row_spec(BR, D),
            _part_spec(HIST_W), _part_spec(HIST_W),
            _full_spec((D, D)), _full_spec((1, D)), _full_spec((D, D)),
        ],
        out_specs=[_part_spec(D), _row_spec(BR, D)],
        out_shape=[ym_shape, jax.ShapeDtypeStruct((N, D), jnp.float32)],
    )(p1, ym1, xa1, degp, cntp, lin2_W, lin2_b.reshape(1, D), agg2_W)

    p2 = _scatter_kernel()(src_s, dst_p, ym2, z_d)

    out = pl.pallas_call(
        _tc_c_body,
        grid=grid,
        in_specs=[
            _part_spec(D), _part_spec(D), _row_spec(BR, D),
            _part_spec(HIST_W), _part_spec(HIST_W),
            _full_spec((D, D)), _full_spec((1, D)),
            _full_spec((D, OUT)), _full_spec((1, OUT)),
        ],
        out_specs=_row_spec(BR, OUT),
        out_shape=jax.ShapeDtypeStruct((N, OUT), jnp.float32),
    )(p2, ym2, xa2, degp, cntp, mp1_W, mp1_b.reshape(1, D),
      mp2_W, mp2_b.reshape(1, OUT))

    return out


# speedup vs baseline: 1.0563x; 1.0563x over previous
import functools

import jax
import jax.numpy as jnp
from jax import lax
from jax.experimental import pallas as pl
from jax.experimental.pallas import tpu as pltpu
from jax.experimental.pallas import tpu_sc as plsc

N = 10000
E = 320000
D = 128
OUT = 64

NC = 2
NS = 16
NW = NC * NS

CHUNK = 128
CPT = 80
HCPT = 40
CPT0 = 120
CPT1 = 40
E_PAD = NW * CPT * CHUNK
N_ACC = 10112
ZROWS = N_ACC // NS
HIST_W = 16

BR = 1000


def _mesh():
    return plsc.VectorSubcoreMesh(core_axis_name="c", subcore_axis_name="s")


@functools.lru_cache(maxsize=None)
def _hist_kernel():
    @functools.partial(
        pl.kernel,
        out_type=(
            jax.ShapeDtypeStruct((NC, N_ACC, HIST_W), jnp.float32),
            jax.ShapeDtypeStruct((NC, N_ACC, HIST_W), jnp.float32),
        ),
        mesh=_mesh(),
        scratch_types=[
            pltpu.VMEM((CPT, CHUNK), jnp.int32),
            pltpu.VMEM((CPT, CHUNK), jnp.int32),
            pltpu.VMEM((CHUNK, HIST_W), jnp.float32),
            pltpu.VMEM_SHARED((N_ACC, HIST_W), jnp.float32),
            pltpu.VMEM_SHARED((N_ACC, HIST_W), jnp.float32),
        ],
    )
    def hist(src_hbm, dst_hbm, ones_hbm, z_hbm, degp_hbm, cntp_hbm,
             src_v, dst_v, ones_v, accd, accc):
        c = lax.axis_index("c")
        s = lax.axis_index("s")
        w = c * NS + s
        pltpu.sync_copy(z_hbm.at[pl.ds(s * ZROWS, ZROWS)],
                        accd.at[pl.ds(s * ZROWS, ZROWS)])
        pltpu.sync_copy(z_hbm.at[pl.ds(s * ZROWS, ZROWS)],
                        accc.at[pl.ds(s * ZROWS, ZROWS)])
        pltpu.sync_copy(ones_hbm, ones_v)
        pltpu.sync_copy(src_hbm.at[pl.ds(w * CPT, CPT)], src_v)
        pltpu.sync_copy(dst_hbm.at[pl.ds(w * CPT, CPT)], dst_v)
        plsc.subcore_barrier()

        def body(j, carry):
            pltpu.sync_copy(ones_v, accd.at[src_v.at[j]], add=True)
            pltpu.sync_copy(ones_v, accc.at[dst_v.at[j]], add=True)
            return carry

        lax.fori_loop(0, CPT, body, 0)
        plsc.subcore_barrier()
        pltpu.sync_copy(accd.at[pl.ds(s * ZROWS, ZROWS)],
                        degp_hbm.at[c, pl.ds(s * ZROWS, ZROWS)])
        pltpu.sync_copy(accc.at[pl.ds(s * ZROWS, ZROWS)],
                        cntp_hbm.at[c, pl.ds(s * ZROWS, ZROWS)])

    return hist


@functools.lru_cache(maxsize=None)
def _scatter_kernel():
    @functools.partial(
        pl.kernel,
        out_type=jax.ShapeDtypeStruct((NC, N_ACC, D), jnp.float32),
        mesh=_mesh(),
        scratch_types=[
            pltpu.VMEM((HCPT, CHUNK), jnp.int32),
            pltpu.VMEM((HCPT, CHUNK), jnp.int32),
            pltpu.VMEM((CHUNK, D), jnp.float32),
            pltpu.VMEM((CHUNK, D), jnp.float32),
            pltpu.VMEM_SHARED((N_ACC, D), jnp.float32),
            pltpu.SemaphoreType.DMA,
            pltpu.SemaphoreType.DMA,
        ],
    )
    def scat(src_hbm, dst_hbm, ym_hbm, z_hbm, out_hbm,
             src_v, dst_v, rows_a, rows_b, acc, sem_a, sem_b):
        c = lax.axis_index("c")
        s = lax.axis_index("s")
        pltpu.sync_copy(z_hbm.at[pl.ds(s * ZROWS, ZROWS)],
                        acc.at[pl.ds(s * ZROWS, ZROWS)])
        plsc.subcore_barrier()

        def body(i, carry):
            j = 2 * i
            da = pltpu.async_copy(ym_hbm.at[src_v.at[j]], rows_a, sem_a)
            db = pltpu.async_copy(ym_hbm.at[src_v.at[j + 1]], rows_b, sem_b)
            da.wait()
            pltpu.sync_copy(rows_a, acc.at[dst_v.at[j]], add=True)
            db.wait()
            pltpu.sync_copy(rows_b, acc.at[dst_v.at[j + 1]], add=True)
            return carry

        def run(base0, nparts):
            for h in range(nparts):
                base = base0 + h * HCPT
                pltpu.sync_copy(src_hbm.at[pl.ds(base, HCPT)], src_v)
                pltpu.sync_copy(dst_hbm.at[pl.ds(base, HCPT)], dst_v)
                lax.fori_loop(0, HCPT // 2, body, 0)

        @pl.when(c == 0)
        def _():
            run(NS * CPT1 + s * CPT0, CPT0 // HCPT)

        @pl.when(c == 1)
        def _():
            run(s * CPT1, CPT1 // HCPT)

        plsc.subcore_barrier()
        pltpu.sync_copy(acc.at[pl.ds(s * ZROWS, ZROWS)],
                        out_hbm.at[c, pl.ds(s * ZROWS, ZROWS)])

    return scat


def _dis_icnt(degp, cntp):
    deg = (degp[0] + degp[1])[:, :1]
    cnt = (cntp[0] + cntp[1])[:, :1]
    return lax.rsqrt(deg), 1.0 / cnt


def _combine(p, ym, xa, dis, icnt):
    s = p[0] + p[1]
    aggr = dis * (s + ym) * icnt
    o = jnp.maximum(aggr + xa, 0.0)
    n2 = jnp.sum(o * o, axis=1, keepdims=True)
    nrm = jnp.maximum(jnp.sqrt(n2), 1e-12)
    return o / nrm


def _tc_a_body(x_ref, W_ref, b_ref, Wa_ref, degp_ref, cntp_ref, ym_ref, xa_ref):
    xb = x_ref[...]
    dis, _ = _dis_icnt(degp_ref[...], cntp_ref[...])
    xm = jnp.maximum(jnp.dot(xb, W_ref[...],
                             preferred_element_type=jnp.float32) + b_ref[...], 0.0)
    ym_ref[...] = xm * dis
    xa_ref[...] = jnp.dot(xb, Wa_ref[...], preferred_element_type=jnp.float32)


def _tc_b_body(p_ref, ym_ref, xa_ref, degp_ref, cntp_ref, W_ref, b_ref, Wa_ref,
               ym2_ref, xa2_ref):
    dis, icnt = _dis_icnt(degp_ref[...], cntp_ref[...])
    h = _combine(p_ref[...], ym_ref[...], xa_ref[...], dis, icnt)
    xm2 = jnp.maximum(jnp.dot(h, W_ref[...],
                              preferred_element_type=jnp.float32) + b_ref[...], 0.0)
    ym2_ref[...] = xm2 * dis
    xa2_ref[...] = jnp.dot(h, Wa_ref[...], preferred_element_type=jnp.float32)


def _tc_c_body(p_ref, ym_ref, xa_ref, degp_ref, cntp_ref,
               W1_ref, b1_ref, W2_ref, b2_ref, out_ref):
    dis, icnt = _dis_icnt(degp_ref[...], cntp_ref[...])
    h = _combine(p_ref[...], ym_ref[...], xa_ref[...], dis, icnt)
    z = jnp.dot(h, W1_ref[...], preferred_element_type=jnp.float32) + b1_ref[...]
    o = jnp.dot(z, W2_ref[...], preferred_element_type=jnp.float32) + b2_ref[...]
    m = jnp.max(o, axis=1, keepdims=True)
    lse = jnp.log(jnp.sum(jnp.exp(o - m), axis=1, keepdims=True)) + m
    out_ref[...] = o - lse


def _row_spec(rows, cols):
    return pl.BlockSpec((rows, cols), lambda i: (i, 0))


def _full_spec(shape):
    ndim = len(shape)
    return pl.BlockSpec(shape, lambda i, _n=ndim: (0,) * _n)


def _part_spec(width):
    return pl.BlockSpec((NC, BR, width), lambda i: (0, i, 0))


def kernel(x, edge_index, batch, lin1_W, lin1_b, agg1_W, lin2_W, lin2_b,
           agg2_W, mp1_W, mp1_b, mp2_W, mp2_b):
    src = edge_index[0]
    dst = edge_index[1]
    pad = E_PAD - E
    scratch_idx = N + jnp.arange(pad, dtype=jnp.int32) % (N_ACC - N)
    dst_p = jnp.concatenate([dst, scratch_idx]).reshape(NW * CPT, CHUNK)
    src_h = jnp.concatenate([src, scratch_idx]).reshape(NW * CPT, CHUNK)
    src_s = jnp.concatenate([src, jnp.zeros((pad,), jnp.int32)]).reshape(
        NW * CPT, CHUNK)
    ones_h = jnp.ones((CHUNK, HIST_W), jnp.float32)
    z_h = jnp.zeros((N_ACC, HIST_W), jnp.float32)
    z_d = jnp.zeros((N_ACC, D), jnp.float32)

    degp, cntp = _hist_kernel()(src_h, dst_p, ones_h, z_h)

    ym_shape = jax.ShapeDtypeStruct((N, D), jnp.float32)
    grid = (N // BR,)
    ym1, xa1 = pl.pallas_call(
        _tc_a_body,
        grid=grid,
        in_specs=[
            _row_spec(BR, D), _full_spec((D, D)), _full_spec((1, D)),
            _full_spec((D, D)), _part_spec(HIST_W), _part_spec(HIST_W),
        ],
        out_specs=[_row_spec(BR, D), _row_spec(BR, D)],
        out_shape=[ym_shape, jax.ShapeDtypeStruct((N, D), jnp.float32)],
    )(x, lin1_W, lin1_b.reshape(1, D), agg1_W, degp, cntp)

    p1 = _scatter_kernel()(src_s, dst_p, ym1, z_d)

    ym2, xa2 = pl.pallas_call(
        _tc_b_body,
        grid=grid,
        in_specs=[
            _part_spec(D), _row_spec(BR, D), _row_spec(BR, D),
            _part_spec(HIST_W), _part_spec(HIST_W),
            _full_spec((D, D)), _full_spec((1, D)), _full_spec((D, D)),
        ],
        out_specs=[_row_spec(BR, D), _row_spec(BR, D)],
        out_shape=[ym_shape, jax.ShapeDtypeStruct((N, D), jnp.float32)],
    )(p1, ym1, xa1, degp, cntp, lin2_W, lin2_b.reshape(1, D), agg2_W)

    p2 = _scatter_kernel()(src_s, dst_p, ym2, z_d)

    out = pl.pallas_call(
        _tc_c_body,
        grid=grid,
        in_specs=[
            _part_spec(D), _row_spec(BR, D), _row_spec(BR, D),
            _part_spec(HIST_W), _part_spec(HIST_W),
            _full_spec((D, D)), _full_spec((1, D)),
            _full_spec((D, OUT)), _full_spec((1, OUT)),
        ],
        out_specs=_row_spec(BR, OUT),
        out_shape=jax.ShapeDtypeStruct((N, OUT), jnp.float32),
    )(p2, ym2, xa2, degp, cntp, mp1_W, mp1_b.reshape(1, D),
      mp2_W, mp2_b.reshape(1, OUT))

    return out


# speedup vs baseline: 2.9856x; 2.8265x over previous
import functools

import jax
import jax.numpy as jnp
from jax import lax
from jax.experimental import pallas as pl
from jax.experimental.pallas import tpu as pltpu
from jax.experimental.pallas import tpu_sc as plsc

N = 10000
E = 320000
D = 128
OUT = 64

NC = 2
NS = 16
NW = NC * NS

CHUNK = 128
CPT = 80
HCPT = 40
E_PAD = NW * CPT * CHUNK
N_ACC = 10112
ZROWS = N_ACC // NS
HIST_W = 16

BR = 1000


def _mesh():
    return plsc.VectorSubcoreMesh(core_axis_name="c", subcore_axis_name="s")


@functools.lru_cache(maxsize=None)
def _hist_kernel():
    @functools.partial(
        pl.kernel,
        out_type=(
            jax.ShapeDtypeStruct((NC, N_ACC, HIST_W), jnp.float32),
            jax.ShapeDtypeStruct((NC, N_ACC, HIST_W), jnp.float32),
        ),
        mesh=_mesh(),
        scratch_types=[
            pltpu.VMEM((CPT, CHUNK), jnp.int32),
            pltpu.VMEM((CPT, CHUNK), jnp.int32),
            pltpu.VMEM((CHUNK, HIST_W), jnp.float32),
            pltpu.VMEM_SHARED((N_ACC, HIST_W), jnp.float32),
            pltpu.VMEM_SHARED((N_ACC, HIST_W), jnp.float32),
        ],
    )
    def hist(src_hbm, dst_hbm, ones_hbm, z_hbm, degp_hbm, cntp_hbm,
             src_v, dst_v, ones_v, accd, accc):
        c = lax.axis_index("c")
        s = lax.axis_index("s")
        w = c * NS + s
        pltpu.sync_copy(z_hbm.at[pl.ds(s * ZROWS, ZROWS)],
                        accd.at[pl.ds(s * ZROWS, ZROWS)])
        pltpu.sync_copy(z_hbm.at[pl.ds(s * ZROWS, ZROWS)],
                        accc.at[pl.ds(s * ZROWS, ZROWS)])
        pltpu.sync_copy(ones_hbm, ones_v)
        pltpu.sync_copy(src_hbm.at[pl.ds(w * CPT, CPT)], src_v)
        pltpu.sync_copy(dst_hbm.at[pl.ds(w * CPT, CPT)], dst_v)
        plsc.subcore_barrier()

        def body(j, carry):
            pltpu.sync_copy(ones_v, accd.at[src_v.at[j]], add=True)
            pltpu.sync_copy(ones_v, accc.at[dst_v.at[j]], add=True)
            return carry

        lax.fori_loop(0, CPT, body, 0)
        plsc.subcore_barrier()
        pltpu.sync_copy(accd.at[pl.ds(s * ZROWS, ZROWS)],
                        degp_hbm.at[c, pl.ds(s * ZROWS, ZROWS)])
        pltpu.sync_copy(accc.at[pl.ds(s * ZROWS, ZROWS)],
                        cntp_hbm.at[c, pl.ds(s * ZROWS, ZROWS)])

    return hist


@functools.lru_cache(maxsize=None)
def _scatter_kernel():
    @functools.partial(
        pl.kernel,
        out_type=jax.ShapeDtypeStruct((NC, N_ACC, D), jnp.float32),
        mesh=_mesh(),
        scratch_types=[
            pltpu.VMEM((HCPT, CHUNK), jnp.int32),
            pltpu.VMEM((HCPT, CHUNK), jnp.int32),
            pltpu.VMEM((CHUNK, D), jnp.float32),
            pltpu.VMEM((CHUNK, D), jnp.float32),
            pltpu.VMEM_SHARED((N_ACC, D), jnp.float32),
            pltpu.SemaphoreType.DMA,
            pltpu.SemaphoreType.DMA,
        ],
    )
    def scat(src_hbm, dst_hbm, ym_hbm, z_hbm, out_hbm,
             src_v, dst_v, rows_a, rows_b, acc, sem_a, sem_b):
        c = lax.axis_index("c")
        s = lax.axis_index("s")
        pltpu.sync_copy(z_hbm.at[pl.ds(s * ZROWS, ZROWS)],
                        acc.at[pl.ds(s * ZROWS, ZROWS)])
        plsc.subcore_barrier()

        def body(i, carry):
            j = 2 * i
            da = pltpu.async_copy(ym_hbm.at[src_v.at[j]], rows_a, sem_a)
            db = pltpu.async_copy(ym_hbm.at[src_v.at[j + 1]], rows_b, sem_b)
            da.wait()
            pltpu.sync_copy(rows_a, acc.at[dst_v.at[j]], add=True)
            db.wait()
            pltpu.sync_copy(rows_b, acc.at[dst_v.at[j + 1]], add=True)
            return carry

        w = c * NS + s
        for h in range(CPT // HCPT):
            base = w * CPT + h * HCPT
            pltpu.sync_copy(src_hbm.at[pl.ds(base, HCPT)], src_v)
            pltpu.sync_copy(dst_hbm.at[pl.ds(base, HCPT)], dst_v)
            lax.fori_loop(0, HCPT // 2, body, 0)

        plsc.subcore_barrier()
        pltpu.sync_copy(acc.at[pl.ds(s * ZROWS, ZROWS)],
                        out_hbm.at[c, pl.ds(s * ZROWS, ZROWS)])

    return scat


def _dis_icnt(degp, cntp):
    deg = (degp[0] + degp[1])[:, :1]
    cnt = (cntp[0] + cntp[1])[:, :1]
    return lax.rsqrt(deg), 1.0 / cnt


def _combine(p, ym, xa, dis, icnt):
    s = p[0] + p[1]
    aggr = dis * (s + ym) * icnt
    o = jnp.maximum(aggr + xa, 0.0)
    n2 = jnp.sum(o * o, axis=1, keepdims=True)
    nrm = jnp.maximum(jnp.sqrt(n2), 1e-12)
    return o / nrm


def _tc_a_body(x_ref, W_ref, b_ref, Wa_ref, degp_ref, cntp_ref, ym_ref, xa_ref):
    xb = x_ref[...]
    dis, _ = _dis_icnt(degp_ref[...], cntp_ref[...])
    xm = jnp.maximum(jnp.dot(xb, W_ref[...],
                             preferred_element_type=jnp.float32) + b_ref[...], 0.0)
    ym_ref[...] = xm * dis
    xa_ref[...] = jnp.dot(xb, Wa_ref[...], preferred_element_type=jnp.float32)


def _tc_b_body(p_ref, ym_ref, xa_ref, degp_ref, cntp_ref, W_ref, b_ref, Wa_ref,
               ym2_ref, xa2_ref):
    dis, icnt = _dis_icnt(degp_ref[...], cntp_ref[...])
    h = _combine(p_ref[...], ym_ref[...], xa_ref[...], dis, icnt)
    xm2 = jnp.maximum(jnp.dot(h, W_ref[...],
                              preferred_element_type=jnp.float32) + b_ref[...], 0.0)
    ym2_ref[...] = xm2 * dis
    xa2_ref[...] = jnp.dot(h, Wa_ref[...], preferred_element_type=jnp.float32)


def _tc_c_body(p_ref, ym_ref, xa_ref, degp_ref, cntp_ref,
               W1_ref, b1_ref, W2_ref, b2_ref, out_ref):
    dis, icnt = _dis_icnt(degp_ref[...], cntp_ref[...])
    h = _combine(p_ref[...], ym_ref[...], xa_ref[...], dis, icnt)
    z = jnp.dot(h, W1_ref[...], preferred_element_type=jnp.float32) + b1_ref[...]
    o = jnp.dot(z, W2_ref[...], preferred_element_type=jnp.float32) + b2_ref[...]
    m = jnp.max(o, axis=1, keepdims=True)
    lse = jnp.log(jnp.sum(jnp.exp(o - m), axis=1, keepdims=True)) + m
    out_ref[...] = o - lse


def _row_spec(rows, cols):
    return pl.BlockSpec((rows, cols), lambda i: (i, 0))


def _full_spec(shape):
    ndim = len(shape)
    return pl.BlockSpec(shape, lambda i, _n=ndim: (0,) * _n)


def _part_spec(width):
    return pl.BlockSpec((NC, BR, width), lambda i: (0, i, 0))


def kernel(x, edge_index, batch, lin1_W, lin1_b, agg1_W, lin2_W, lin2_b,
           agg2_W, mp1_W, mp1_b, mp2_W, mp2_b):
    src = edge_index[0]
    dst = edge_index[1]
    pad = E_PAD - E
    iota_pad = jnp.arange(pad, dtype=jnp.int32)
    scratch_idx = N + iota_pad % (N_ACC - N)
    dst_p = jnp.concatenate([dst, scratch_idx]).reshape(NW * CPT, CHUNK)
    src_h = jnp.concatenate([src, scratch_idx]).reshape(NW * CPT, CHUNK)
    src_s = jnp.concatenate([src, iota_pad % N]).reshape(NW * CPT, CHUNK)
    ones_h = jnp.ones((CHUNK, HIST_W), jnp.float32)
    z_h = jnp.zeros((N_ACC, HIST_W), jnp.float32)
    z_d = jnp.zeros((N_ACC, D), jnp.float32)

    degp, cntp = _hist_kernel()(src_h, dst_p, ones_h, z_h)

    ym_shape = jax.ShapeDtypeStruct((N, D), jnp.float32)
    grid = (N // BR,)
    ym1, xa1 = pl.pallas_call(
        _tc_a_body,
        grid=grid,
        in_specs=[
            _row_spec(BR, D), _full_spec((D, D)), _full_spec((1, D)),
            _full_spec((D, D)), _part_spec(HIST_W), _part_spec(HIST_W),
        ],
        out_specs=[_row_spec(BR, D), _row_spec(BR, D)],
        out_shape=[ym_shape, jax.ShapeDtypeStruct((N, D), jnp.float32)],
    )(x, lin1_W, lin1_b.reshape(1, D), agg1_W, degp, cntp)

    p1 = _scatter_kernel()(src_s, dst_p, ym1, z_d)

    ym2, xa2 = pl.pallas_call(
        _tc_b_body,
        grid=grid,
        in_specs=[
            _part_spec(D), _row_spec(BR, D), _row_spec(BR, D),
            _part_spec(HIST_W), _part_spec(HIST_W),
            _full_spec((D, D)), _full_spec((1, D)), _full_spec((D, D)),
        ],
        out_specs=[_row_spec(BR, D), _row_spec(BR, D)],
        out_shape=[ym_shape, jax.ShapeDtypeStruct((N, D), jnp.float32)],
    )(p1, ym1, xa1, degp, cntp, lin2_W, lin2_b.reshape(1, D), agg2_W)

    p2 = _scatter_kernel()(src_s, dst_p, ym2, z_d)

    out = pl.pallas_call(
        _tc_c_body,
        grid=grid,
        in_specs=[
            _part_spec(D), _row_spec(BR, D), _row_spec(BR, D),
            _part_spec(HIST_W), _part_spec(HIST_W),
            _full_spec((D, D)), _full_spec((1, D)),
            _full_spec((D, OUT)), _full_spec((1, OUT)),
        ],
        out_specs=_row_spec(BR, OUT),
        out_shape=jax.ShapeDtypeStruct((N, OUT), jnp.float32),
    )(p2, ym2, xa2, degp, cntp, mp1_W, mp1_b.reshape(1, D),
      mp2_W, mp2_b.reshape(1, OUT))

    return out


# speedup vs baseline: 3.0735x; 1.0294x over previous
import functools

import jax
import jax.numpy as jnp
from jax import lax
from jax.experimental import pallas as pl
from jax.experimental.pallas import tpu as pltpu
from jax.experimental.pallas import tpu_sc as plsc

N = 10000
E = 320000
D = 128
OUT = 64

NC = 2
NS = 16
NW = NC * NS

CHUNK = 128
CPT = 80
HCPT = 40
E_PAD = NW * CPT * CHUNK
N_ACC = 10112
ZROWS = N_ACC // NS
HIST_W = 16

BR = 1000


def _mesh():
    return plsc.VectorSubcoreMesh(core_axis_name="c", subcore_axis_name="s")


@functools.lru_cache(maxsize=None)
def _hist_kernel():
    @functools.partial(
        pl.kernel,
        out_type=(
            jax.ShapeDtypeStruct((NC, N_ACC, HIST_W), jnp.float32),
            jax.ShapeDtypeStruct((NC, N_ACC, HIST_W), jnp.float32),
        ),
        mesh=_mesh(),
        scratch_types=[
            pltpu.VMEM((CPT, CHUNK), jnp.int32),
            pltpu.VMEM((CPT, CHUNK), jnp.int32),
            pltpu.VMEM((CHUNK, HIST_W), jnp.float32),
            pltpu.VMEM_SHARED((N_ACC, HIST_W), jnp.float32),
            pltpu.VMEM_SHARED((N_ACC, HIST_W), jnp.float32),
            pltpu.SemaphoreType.DMA,
            pltpu.SemaphoreType.DMA,
        ],
    )
    def hist(src_hbm, dst_hbm, ones_hbm, z_hbm, degp_hbm, cntp_hbm,
             src_v, dst_v, ones_v, accd, accc, sem_d, sem_c):
        c = lax.axis_index("c")
        s = lax.axis_index("s")
        w = c * NS + s
        pltpu.sync_copy(z_hbm.at[pl.ds(s * ZROWS, ZROWS)],
                        accd.at[pl.ds(s * ZROWS, ZROWS)])
        pltpu.sync_copy(z_hbm.at[pl.ds(s * ZROWS, ZROWS)],
                        accc.at[pl.ds(s * ZROWS, ZROWS)])
        pltpu.sync_copy(ones_hbm, ones_v)
        pltpu.sync_copy(src_hbm.at[pl.ds(w * CPT, CPT)], src_v)
        pltpu.sync_copy(dst_hbm.at[pl.ds(w * CPT, CPT)], dst_v)
        plsc.subcore_barrier()

        def body(j, carry):
            @pl.when(j >= 2)
            def _():
                pltpu.make_async_copy(ones_v, accd.at[src_v.at[j - 2]],
                                      sem_d).wait()
                pltpu.make_async_copy(ones_v, accc.at[dst_v.at[j - 2]],
                                      sem_c).wait()

            pltpu.make_async_copy(ones_v, accd.at[src_v.at[j]],
                                  sem_d).start(add=True)
            pltpu.make_async_copy(ones_v, accc.at[dst_v.at[j]],
                                  sem_c).start(add=True)
            return carry

        lax.fori_loop(0, CPT, body, 0)
        for j in (CPT - 2, CPT - 1):
            pltpu.make_async_copy(ones_v, accd.at[src_v.at[j]], sem_d).wait()
            pltpu.make_async_copy(ones_v, accc.at[dst_v.at[j]], sem_c).wait()
        plsc.subcore_barrier()
        pltpu.sync_copy(accd.at[pl.ds(s * ZROWS, ZROWS)],
                        degp_hbm.at[c, pl.ds(s * ZROWS, ZROWS)])
        pltpu.sync_copy(accc.at[pl.ds(s * ZROWS, ZROWS)],
                        cntp_hbm.at[c, pl.ds(s * ZROWS, ZROWS)])

    return hist


@functools.lru_cache(maxsize=None)
def _scatter_kernel():
    @functools.partial(
        pl.kernel,
        out_type=jax.ShapeDtypeStruct((NC, N_ACC, D), jnp.float32),
        mesh=_mesh(),
        scratch_types=[
            pltpu.VMEM((HCPT, CHUNK), jnp.int32),
            pltpu.VMEM((HCPT, CHUNK), jnp.int32),
            pltpu.VMEM((CHUNK, D), jnp.float32),
            pltpu.VMEM((CHUNK, D), jnp.float32),
            pltpu.VMEM_SHARED((N_ACC, D), jnp.float32),
            pltpu.SemaphoreType.DMA,
            pltpu.SemaphoreType.DMA,
            pltpu.SemaphoreType.DMA,
            pltpu.SemaphoreType.DMA,
        ],
    )
    def scat(src_hbm, dst_hbm, ym_hbm, z_hbm, out_hbm,
             src_v, dst_v, rows_a, rows_b, acc, sem_a, sem_b, sem_c, sem_d):
        c = lax.axis_index("c")
        s = lax.axis_index("s")
        pltpu.sync_copy(z_hbm.at[pl.ds(s * ZROWS, ZROWS)],
                        acc.at[pl.ds(s * ZROWS, ZROWS)])
        plsc.subcore_barrier()

        def body(i, carry):
            j = 2 * i

            @pl.when(i > 0)
            def _():
                pltpu.make_async_copy(rows_a, acc.at[dst_v.at[j - 2]],
                                      sem_c).wait()
                pltpu.make_async_copy(rows_b, acc.at[dst_v.at[j - 1]],
                                      sem_d).wait()

            ga = pltpu.async_copy(ym_hbm.at[src_v.at[j]], rows_a, sem_a)
            gb = pltpu.async_copy(ym_hbm.at[src_v.at[j + 1]], rows_b, sem_b)
            ga.wait()
            pltpu.make_async_copy(rows_a, acc.at[dst_v.at[j]],
                                  sem_c).start(add=True)
            gb.wait()
            pltpu.make_async_copy(rows_b, acc.at[dst_v.at[j + 1]],
                                  sem_d).start(add=True)
            return carry

        w = c * NS + s
        for h in range(CPT // HCPT):
            base = w * CPT + h * HCPT
            pltpu.sync_copy(src_hbm.at[pl.ds(base, HCPT)], src_v)
            pltpu.sync_copy(dst_hbm.at[pl.ds(base, HCPT)], dst_v)
            lax.fori_loop(0, HCPT // 2, body, 0)
            pltpu.make_async_copy(rows_a, acc.at[dst_v.at[HCPT - 2]],
                                  sem_c).wait()
            pltpu.make_async_copy(rows_b, acc.at[dst_v.at[HCPT - 1]],
                                  sem_d).wait()

        plsc.subcore_barrier()
        pltpu.sync_copy(acc.at[pl.ds(s * ZROWS, ZROWS)],
                        out_hbm.at[c, pl.ds(s * ZROWS, ZROWS)])

    return scat


def _dis_icnt(degp, cntp):
    deg = (degp[0] + degp[1])[:, :1]
    cnt = (cntp[0] + cntp[1])[:, :1]
    return lax.rsqrt(deg), 1.0 / cnt


def _combine(p, ym, xa, dis, icnt):
    s = p[0] + p[1]
    aggr = dis * (s + ym) * icnt
    o = jnp.maximum(aggr + xa, 0.0)
    n2 = jnp.sum(o * o, axis=1, keepdims=True)
    nrm = jnp.maximum(jnp.sqrt(n2), 1e-12)
    return o / nrm


def _tc_a_body(x_ref, W_ref, b_ref, Wa_ref, degp_ref, cntp_ref, ym_ref, xa_ref):
    xb = x_ref[...]
    dis, _ = _dis_icnt(degp_ref[...], cntp_ref[...])
    xm = jnp.maximum(jnp.dot(xb, W_ref[...],
                             preferred_element_type=jnp.float32) + b_ref[...], 0.0)
    ym_ref[...] = xm * dis
    xa_ref[...] = jnp.dot(xb, Wa_ref[...], preferred_element_type=jnp.float32)


def _tc_b_body(p_ref, ym_ref, xa_ref, degp_ref, cntp_ref, W_ref, b_ref, Wa_ref,
               ym2_ref, xa2_ref):
    dis, icnt = _dis_icnt(degp_ref[...], cntp_ref[...])
    h = _combine(p_ref[...], ym_ref[...], xa_ref[...], dis, icnt)
    xm2 = jnp.maximum(jnp.dot(h, W_ref[...],
                              preferred_element_type=jnp.float32) + b_ref[...], 0.0)
    ym2_ref[...] = xm2 * dis
    xa2_ref[...] = jnp.dot(h, Wa_ref[...], preferred_element_type=jnp.float32)


def _tc_c_body(p_ref, ym_ref, xa_ref, degp_ref, cntp_ref,
               W1_ref, b1_ref, W2_ref, b2_ref, out_ref):
    dis, icnt = _dis_icnt(degp_ref[...], cntp_ref[...])
    h = _combine(p_ref[...], ym_ref[...], xa_ref[...], dis, icnt)
    z = jnp.dot(h, W1_ref[...], preferred_element_type=jnp.float32) + b1_ref[...]
    o = jnp.dot(z, W2_ref[...], preferred_element_type=jnp.float32) + b2_ref[...]
    m = jnp.max(o, axis=1, keepdims=True)
    lse = jnp.log(jnp.sum(jnp.exp(o - m), axis=1, keepdims=True)) + m
    out_ref[...] = o - lse


def _row_spec(rows, cols):
    return pl.BlockSpec((rows, cols), lambda i: (i, 0))


def _full_spec(shape):
    ndim = len(shape)
    return pl.BlockSpec(shape, lambda i, _n=ndim: (0,) * _n)


def _part_spec(width):
    return pl.BlockSpec((NC, BR, width), lambda i: (0, i, 0))


def kernel(x, edge_index, batch, lin1_W, lin1_b, agg1_W, lin2_W, lin2_b,
           agg2_W, mp1_W, mp1_b, mp2_W, mp2_b):
    src = edge_index[0]
    dst = edge_index[1]
    pad = E_PAD - E
    iota_pad = jnp.arange(pad, dtype=jnp.int32)
    scratch_idx = N + iota_pad % (N_ACC - N)
    dst_p = jnp.concatenate([dst, scratch_idx]).reshape(NW * CPT, CHUNK)
    src_h = jnp.concatenate([src, scratch_idx]).reshape(NW * CPT, CHUNK)
    src_s = jnp.concatenate([src, iota_pad % N]).reshape(NW * CPT, CHUNK)
    ones_h = jnp.ones((CHUNK, HIST_W), jnp.float32)
    z_h = jnp.zeros((N_ACC, HIST_W), jnp.float32)
    z_d = jnp.zeros((N_ACC, D), jnp.float32)

    degp, cntp = _hist_kernel()(src_h, dst_p, ones_h, z_h)

    ym_shape = jax.ShapeDtypeStruct((N, D), jnp.float32)
    grid = (N // BR,)
    ym1, xa1 = pl.pallas_call(
        _tc_a_body,
        grid=grid,
        in_specs=[
            _row_spec(BR, D), _full_spec((D, D)), _full_spec((1, D)),
            _full_spec((D, D)), _part_spec(HIST_W), _part_spec(HIST_W),
        ],
        out_specs=[_row_spec(BR, D), _row_spec(BR, D)],
        out_shape=[ym_shape, jax.ShapeDtypeStruct((N, D), jnp.float32)],
    )(x, lin1_W, lin1_b.reshape(1, D), agg1_W, degp, cntp)

    p1 = _scatter_kernel()(src_s, dst_p, ym1, z_d)

    ym2, xa2 = pl.pallas_call(
        _tc_b_body,
        grid=grid,
        in_specs=[
            _part_spec(D), _row_spec(BR, D), _row_spec(BR, D),
            _part_spec(HIST_W), _part_spec(HIST_W),
            _full_spec((D, D)), _full_spec((1, D)), _full_spec((D, D)),
        ],
        out_specs=[_row_spec(BR, D), _row_spec(BR, D)],
        out_shape=[ym_shape, jax.ShapeDtypeStruct((N, D), jnp.float32)],
    )(p1, ym1, xa1, degp, cntp, lin2_W, lin2_b.reshape(1, D), agg2_W)

    p2 = _scatter_kernel()(src_s, dst_p, ym2, z_d)

    out = pl.pallas_call(
        _tc_c_body,
        grid=grid,
        in_specs=[
            _part_spec(D), _row_spec(BR, D), _row_spec(BR, D),
            _part_spec(HIST_W), _part_spec(HIST_W),
            _full_spec((D, D)), _full_spec((1, D)),
            _full_spec((D, OUT)), _full_spec((1, OUT)),
        ],
        out_specs=_row_spec(BR, OUT),
        out_shape=jax.ShapeDtypeStruct((N, OUT), jnp.float32),
    )(p2, ym2, xa2, degp, cntp, mp1_W, mp1_b.reshape(1, D),
      mp2_W, mp2_b.reshape(1, OUT))

    return out


# speedup vs baseline: 3.1155x; 1.0137x over previous
import functools

import numpy as np

import jax
import jax.numpy as jnp
from jax import lax
from jax.experimental import pallas as pl
from jax.experimental.pallas import tpu as pltpu
from jax.experimental.pallas import tpu_sc as plsc

N = 10000
E = 320000
D = 128
OUT = 64

NC = 2
NS = 16
NW = NC * NS

CHUNK = 128
CPT = 80
HCPT = 40
RCH = E // CHUNK
TAILW = NW - 1
E_PAD = NW * CPT * CHUNK
N_ACC = 10112
ZROWS = N_ACC // NS
HIST_W = 16

BR = 1000

NPAD = NW * CPT - RCH
_PAD_IOTA = np.arange(NPAD * CHUNK, dtype=np.int32)
PAD_SCRATCH = N + _PAD_IOTA % (N_ACC - N)
PAD_REAL = _PAD_IOTA % N


def _mesh():
    return plsc.VectorSubcoreMesh(core_axis_name="c", subcore_axis_name="s")


@functools.lru_cache(maxsize=None)
def _hist_kernel():
    @functools.partial(
        pl.kernel,
        out_type=(
            jax.ShapeDtypeStruct((NC, N_ACC, HIST_W), jnp.float32),
            jax.ShapeDtypeStruct((NC, N_ACC, HIST_W), jnp.float32),
        ),
        mesh=_mesh(),
        scratch_types=[
            pltpu.VMEM((HCPT, CHUNK), jnp.int32),
            pltpu.VMEM((HCPT, CHUNK), jnp.int32),
            pltpu.VMEM((CHUNK, HIST_W), jnp.float32),
            pltpu.VMEM_SHARED((N_ACC, HIST_W), jnp.float32),
            pltpu.VMEM_SHARED((N_ACC, HIST_W), jnp.float32),
            pltpu.SemaphoreType.DMA,
            pltpu.SemaphoreType.DMA,
        ],
    )
    def hist(ei_hbm, tsrc_hbm, tdst_hbm, ones_hbm, z_hbm, degp_hbm, cntp_hbm,
             src_v, dst_v, ones_v, accd, accc, sem_d, sem_c):
        c = lax.axis_index("c")
        s = lax.axis_index("s")
        w = c * NS + s
        pltpu.sync_copy(z_hbm.at[pl.ds(s * ZROWS, ZROWS)],
                        accd.at[pl.ds(s * ZROWS, ZROWS)])
        pltpu.sync_copy(z_hbm.at[pl.ds(s * ZROWS, ZROWS)],
                        accc.at[pl.ds(s * ZROWS, ZROWS)])
        pltpu.sync_copy(ones_hbm, ones_v)
        plsc.subcore_barrier()

        def body(j, carry):
            @pl.when(j >= 2)
            def _():
                pltpu.make_async_copy(ones_v, accd.at[src_v.at[j - 2]],
                                      sem_d).wait()
                pltpu.make_async_copy(ones_v, accc.at[dst_v.at[j - 2]],
                                      sem_c).wait()

            pltpu.make_async_copy(ones_v, accd.at[src_v.at[j]],
                                  sem_d).start(add=True)
            pltpu.make_async_copy(ones_v, accc.at[dst_v.at[j]],
                                  sem_c).start(add=True)
            return carry

        for h in range(CPT // HCPT):
            base = w * CPT + h * HCPT

            @pl.when(w < TAILW)
            def _():
                pltpu.sync_copy(ei_hbm.at[0, pl.ds(base, HCPT)], src_v)
                pltpu.sync_copy(ei_hbm.at[1, pl.ds(base, HCPT)], dst_v)

            @pl.when(w == TAILW)
            def _():
                pltpu.sync_copy(tsrc_hbm.at[pl.ds(h * HCPT, HCPT)], src_v)
                pltpu.sync_copy(tdst_hbm.at[pl.ds(h * HCPT, HCPT)], dst_v)

            lax.fori_loop(0, HCPT, body, 0)
            for j in (HCPT - 2, HCPT - 1):
                pltpu.make_async_copy(ones_v, accd.at[src_v.at[j]],
                                      sem_d).wait()
                pltpu.make_async_copy(ones_v, accc.at[dst_v.at[j]],
                                      sem_c).wait()
        plsc.subcore_barrier()
        pltpu.sync_copy(accd.at[pl.ds(s * ZROWS, ZROWS)],
                        degp_hbm.at[c, pl.ds(s * ZROWS, ZROWS)])
        pltpu.sync_copy(accc.at[pl.ds(s * ZROWS, ZROWS)],
                        cntp_hbm.at[c, pl.ds(s * ZROWS, ZROWS)])

    return hist


@functools.lru_cache(maxsize=None)
def _scatter_kernel():
    @functools.partial(
        pl.kernel,
        out_type=jax.ShapeDtypeStruct((NC, N_ACC, D), jnp.float32),
        mesh=_mesh(),
        scratch_types=[
            pltpu.VMEM((HCPT, CHUNK), jnp.int32),
            pltpu.VMEM((HCPT, CHUNK), jnp.int32),
            pltpu.VMEM((CHUNK, D), jnp.float32),
            pltpu.VMEM((CHUNK, D), jnp.float32),
            pltpu.VMEM_SHARED((N_ACC, D), jnp.float32),
            pltpu.SemaphoreType.DMA,
            pltpu.SemaphoreType.DMA,
            pltpu.SemaphoreType.DMA,
            pltpu.SemaphoreType.DMA,
        ],
    )
    def scat(ei_hbm, tsrc_hbm, tdst_hbm, ym_hbm, z_hbm, out_hbm,
             src_v, dst_v, rows_a, rows_b, acc, sem_a, sem_b, sem_c, sem_d):
        c = lax.axis_index("c")
        s = lax.axis_index("s")
        pltpu.sync_copy(z_hbm.at[pl.ds(s * ZROWS, ZROWS)],
                        acc.at[pl.ds(s * ZROWS, ZROWS)])
        plsc.subcore_barrier()

        def body(i, carry):
            j = 2 * i

            @pl.when(i > 0)
            def _():
                pltpu.make_async_copy(rows_a, acc.at[dst_v.at[j - 2]],
                                      sem_c).wait()
                pltpu.make_async_copy(rows_b, acc.at[dst_v.at[j - 1]],
                                      sem_d).wait()

            ga = pltpu.async_copy(ym_hbm.at[src_v.at[j]], rows_a, sem_a)
            gb = pltpu.async_copy(ym_hbm.at[src_v.at[j + 1]], rows_b, sem_b)
            ga.wait()
            pltpu.make_async_copy(rows_a, acc.at[dst_v.at[j]],
                                  sem_c).start(add=True)
            gb.wait()
            pltpu.make_async_copy(rows_b, acc.at[dst_v.at[j + 1]],
                                  sem_d).start(add=True)
            return carry

        w = c * NS + s
        for h in range(CPT // HCPT):
            base = w * CPT + h * HCPT

            @pl.when(w < TAILW)
            def _():
                pltpu.sync_copy(ei_hbm.at[0, pl.ds(base, HCPT)], src_v)
                pltpu.sync_copy(ei_hbm.at[1, pl.ds(base, HCPT)], dst_v)

            @pl.when(w == TAILW)
            def _():
                pltpu.sync_copy(tsrc_hbm.at[pl.ds(h * HCPT, HCPT)], src_v)
                pltpu.sync_copy(tdst_hbm.at[pl.ds(h * HCPT, HCPT)], dst_v)

            lax.fori_loop(0, HCPT // 2, body, 0)
            pltpu.make_async_copy(rows_a, acc.at[dst_v.at[HCPT - 2]],
                                  sem_c).wait()
            pltpu.make_async_copy(rows_b, acc.at[dst_v.at[HCPT - 1]],
                                  sem_d).wait()

        plsc.subcore_barrier()
        pltpu.sync_copy(acc.at[pl.ds(s * ZROWS, ZROWS)],
                        out_hbm.at[c, pl.ds(s * ZROWS, ZROWS)])

    return scat


def _dis_icnt(degp, cntp):
    deg = (degp[0] + degp[1])[:, :1]
    cnt = (cntp[0] + cntp[1])[:, :1]
    return lax.rsqrt(deg), 1.0 / cnt


def _combine(p, ym, xa, dis, icnt):
    s = p[0] + p[1]
    aggr = dis * (s + ym) * icnt
    o = jnp.maximum(aggr + xa, 0.0)
    n2 = jnp.sum(o * o, axis=1, keepdims=True)
    nrm = jnp.maximum(jnp.sqrt(n2), 1e-12)
    return o / nrm


def _tc_a_body(x_ref, W_ref, b_ref, Wa_ref, degp_ref, cntp_ref, ym_ref, xa_ref):
    xb = x_ref[...]
    dis, _ = _dis_icnt(degp_ref[...], cntp_ref[...])
    xm = jnp.maximum(jnp.dot(xb, W_ref[...],
                             preferred_element_type=jnp.float32) + b_ref[...], 0.0)
    ym_ref[...] = xm * dis
    xa_ref[...] = jnp.dot(xb, Wa_ref[...], preferred_element_type=jnp.float32)


def _tc_b_body(p_ref, ym_ref, xa_ref, degp_ref, cntp_ref, W_ref, b_ref, Wa_ref,
               ym2_ref, xa2_ref):
    dis, icnt = _dis_icnt(degp_ref[...], cntp_ref[...])
    h = _combine(p_ref[...], ym_ref[...], xa_ref[...], dis, icnt)
    xm2 = jnp.maximum(jnp.dot(h, W_ref[...],
                              preferred_element_type=jnp.float32) + b_ref[...], 0.0)
    ym2_ref[...] = xm2 * dis
    xa2_ref[...] = jnp.dot(h, Wa_ref[...], preferred_element_type=jnp.float32)


def _tc_c_body(p_ref, ym_ref, xa_ref, degp_ref, cntp_ref,
               W1_ref, b1_ref, W2_ref, b2_ref, out_ref):
    dis, icnt = _dis_icnt(degp_ref[...], cntp_ref[...])
    h = _combine(p_ref[...], ym_ref[...], xa_ref[...], dis, icnt)
    z = jnp.dot(h, W1_ref[...], preferred_element_type=jnp.float32) + b1_ref[...]
    o = jnp.dot(z, W2_ref[...], preferred_element_type=jnp.float32) + b2_ref[...]
    m = jnp.max(o, axis=1, keepdims=True)
    lse = jnp.log(jnp.sum(jnp.exp(o - m), axis=1, keepdims=True)) + m
    out_ref[...] = o - lse


def _row_spec(rows, cols):
    return pl.BlockSpec((rows, cols), lambda i: (i, 0))


def _full_spec(shape):
    ndim = len(shape)
    return pl.BlockSpec(shape, lambda i, _n=ndim: (0,) * _n)


def _part_spec(width):
    return pl.BlockSpec((NC, BR, width), lambda i: (0, i, 0))


def kernel(x, edge_index, batch, lin1_W, lin1_b, agg1_W, lin2_W, lin2_b,
           agg2_W, mp1_W, mp1_b, mp2_W, mp2_b):
    ei3 = edge_index.reshape(2, RCH, CHUNK)
    tail_src = edge_index[0, TAILW * CPT * CHUNK:]
    tail_dst = edge_index[1, TAILW * CPT * CHUNK:]
    pad_scratch = jnp.asarray(PAD_SCRATCH)
    tdst = jnp.concatenate([tail_dst, pad_scratch]).reshape(CPT, CHUNK)
    tsrc_h = jnp.concatenate([tail_src, pad_scratch]).reshape(CPT, CHUNK)
    tsrc_s = jnp.concatenate([tail_src, jnp.asarray(PAD_REAL)]).reshape(
        CPT, CHUNK)
    ones_h = jnp.ones((CHUNK, HIST_W), jnp.float32)
    z_h = jnp.zeros((N_ACC, HIST_W), jnp.float32)
    z_d = jnp.zeros((N_ACC, D), jnp.float32)

    degp, cntp = _hist_kernel()(ei3, tsrc_h, tdst, ones_h, z_h)

    ym_shape = jax.ShapeDtypeStruct((N, D), jnp.float32)
    grid = (N // BR,)
    ym1, xa1 = pl.pallas_call(
        _tc_a_body,
        grid=grid,
        in_specs=[
            _row_spec(BR, D), _full_spec((D, D)), _full_spec((1, D)),
            _full_spec((D, D)), _part_spec(HIST_W), _part_spec(HIST_W),
        ],
        out_specs=[_row_spec(BR, D), _row_spec(BR, D)],
        out_shape=[ym_shape, jax.ShapeDtypeStruct((N, D), jnp.float32)],
    )(x, lin1_W, lin1_b.reshape(1, D), agg1_W, degp, cntp)

    p1 = _scatter_kernel()(ei3, tsrc_s, tdst, ym1, z_d)

    ym2, xa2 = pl.pallas_call(
        _tc_b_body,
        grid=grid,
        in_specs=[
            _part_spec(D), _row_spec(BR, D), _row_spec(BR, D),
            _part_spec(HIST_W), _part_spec(HIST_W),
            _full_spec((D, D)), _full_spec((1, D)), _full_spec((D, D)),
        ],
        out_specs=[_row_spec(BR, D), _row_spec(BR, D)],
        out_shape=[ym_shape, jax.ShapeDtypeStruct((N, D), jnp.float32)],
    )(p1, ym1, xa1, degp, cntp, lin2_W, lin2_b.reshape(1, D), agg2_W)

    p2 = _scatter_kernel()(ei3, tsrc_s, tdst, ym2, z_d)

    out = pl.pallas_call(
        _tc_c_body,
        grid=grid,
        in_specs=[
            _part_spec(D), _row_spec(BR, D), _row_spec(BR, D),
            _part_spec(HIST_W), _part_spec(HIST_W),
            _full_spec((D, D)), _full_spec((1, D)),
            _full_spec((D, OUT)), _full_spec((1, OUT)),
        ],
        out_specs=_row_spec(BR, OUT),
        out_shape=jax.ShapeDtypeStruct((N, OUT), jnp.float32),
    )(p2, ym2, xa2, degp, cntp, mp1_W, mp1_b.reshape(1, D),
      mp2_W, mp2_b.reshape(1, OUT))

    return out
